# baseline (speedup 1.0000x reference)
import math

import jax
import jax.numpy as jnp
from jax.experimental import pallas as pl
from jax.experimental.pallas import tpu as pltpu

K = 2
N = 10000
NFEAT = 128
NH = 64
NCLASS = 40
NLAYERS = 2
LAMDA = 0.5
ALPHA = 0.1

BM = 400


def _fc_body(x_ref, wt_ref, b_ref, o_ref):
    acc = jnp.dot(x_ref[0], wt_ref[0], preferred_element_type=jnp.float32)
    o_ref[0] = jnp.maximum(acc + b_ref[0], 0.0)


def _fc_stage(x, fc_Wt, fc_b3):
    return pl.pallas_call(
        _fc_body,
        grid=(K,),
        in_specs=[
            pl.BlockSpec((1, N, NFEAT), lambda k: (k, 0, 0)),
            pl.BlockSpec((1, NFEAT, NH), lambda k: (k, 0, 0)),
            pl.BlockSpec((1, 1, NH), lambda k: (k, 0, 0)),
        ],
        out_specs=pl.BlockSpec((1, N, NH), lambda k: (k, 0, 0)),
        out_shape=jax.ShapeDtypeStruct((K, N, NH), jnp.float32),
        compiler_params=pltpu.CompilerParams(
            dimension_semantics=("arbitrary",)),
    )(x, fc_Wt, fc_b3)


def _ortho_body(w_ref, m_ref):
    W0 = w_ref[0]
    W1 = w_ref[1]
    wtw0 = jax.lax.dot_general(W0, W0, (((0,), (0,)), ((), ())),
                               preferred_element_type=jnp.float32)
    wtw1 = jax.lax.dot_general(W1, W1, (((0,), (0,)), ((), ())),
                               preferred_element_type=jnp.float32)
    rows1 = jax.lax.broadcasted_iota(jnp.int32, (NH, 1), 0)
    lanes1 = jax.lax.broadcasted_iota(jnp.int32, (1, NH), 1)
    eye = (rows1 == lanes1).astype(jnp.float32)
    A2 = jnp.concatenate([wtw0 + 1e-4 * eye, wtw1 + 1e-4 * eye], axis=0)
    W2 = jnp.concatenate([W0, W1], axis=0)

    rows2 = jax.lax.broadcasted_iota(jnp.int32, (2 * NH, 1), 0)
    r2 = rows2 % NH
    lanes2 = jax.lax.broadcasted_iota(jnp.int32, (1, NH), 1)
    et_rows = jax.lax.broadcasted_iota(jnp.int32, (2 * NH, 2), 0)
    et_lanes = jax.lax.broadcasted_iota(jnp.int32, (2 * NH, 2), 1)
    e_rows = jax.lax.broadcasted_iota(jnp.int32, (2, 2 * NH), 0)
    e_lanes = jax.lax.broadcasted_iota(jnp.int32, (2, 2 * NH), 1)
    P = (et_lanes == et_rows // NH).astype(jnp.float32)

    def sel(k):
        ek = (jax.lax.broadcasted_iota(jnp.int32, (NH, 1), 0) == k)
        ekr = (lanes2 == k).astype(jnp.float32)
        E = (e_lanes == NH * e_rows + k).astype(jnp.float32)
        return ek.astype(jnp.float32), ekr, E

    def chol_step(k, AL):
        A, L = AL
        ek, ekr, E = sel(k)
        colv = jnp.dot(A, ek, preferred_element_type=jnp.float32)
        rowv = jnp.dot(E, A, preferred_element_type=jnp.float32)
        akk = jnp.dot(E, colv, preferred_element_type=jnp.float32)
        inv = jax.lax.rsqrt(akk)
        srow = jnp.where(lanes2 >= k, rowv * inv, 0.0)
        invcol = jnp.dot(P, inv, preferred_element_type=jnp.float32)
        lcol = jnp.where(r2 >= k, colv * invcol, 0.0)
        lrowx = jnp.dot(P, srow, preferred_element_type=jnp.float32)
        A = A - lcol * lrowx
        L = L + lcol * ekr
        return A, L

    zero = jnp.zeros((2 * NH, NH), jnp.float32)
    _, L2 = jax.lax.fori_loop(0, NH, chol_step, (A2, zero))

    def solve_step(j, X):
        ej, ejr, E = sel(j)
        lrow2 = jnp.dot(E, L2, preferred_element_type=jnp.float32)
        ljj = jnp.dot(E, jnp.dot(L2, ej, preferred_element_type=jnp.float32),
                      preferred_element_type=jnp.float32)
        lrowx = jnp.dot(P, lrow2, preferred_element_type=jnp.float32)
        acc = jnp.sum(X * lrowx, axis=1, keepdims=True)
        wcol = jnp.dot(W2, ej, preferred_element_type=jnp.float32)
        rec = jnp.dot(P, 1.0 / ljj, preferred_element_type=jnp.float32)
        xcol = (wcol - acc) * rec
        return X + xcol * ejr

    X2 = jax.lax.fori_loop(0, NH, solve_step, zero)

    t0 = math.log(LAMDA / 1.0 + 1.0)
    t1 = math.log(LAMDA / 2.0 + 1.0)
    theta = jnp.where(rows2 < NH, jnp.float32(t0), jnp.float32(t1))
    eye2 = (r2 == lanes2).astype(jnp.float32)
    M2 = theta * X2 + (1.0 - theta) * eye2
    m_ref[...] = M2.reshape(NLAYERS, NH, NH)


def _ortho_stage(conv_W):
    return pl.pallas_call(
        _ortho_body,
        in_specs=[pl.BlockSpec((NLAYERS, NH, NH), lambda: (0, 0, 0))],
        out_specs=pl.BlockSpec((NLAYERS, NH, NH), lambda: (0, 0, 0)),
        out_shape=jax.ShapeDtypeStruct((NLAYERS, NH, NH), jnp.float32),
    )(conv_W)


def _layer_body_plain(adj_ref, h_ref, h0_ref, m_ref, o_ref):
    hi = jnp.dot(adj_ref[0], h_ref[...], preferred_element_type=jnp.float32)
    support = (1.0 - ALPHA) * hi + ALPHA * h0_ref[...]
    z = jnp.dot(support, m_ref[...], preferred_element_type=jnp.float32)
    o_ref[...] = jnp.maximum(jnp.tanh(z), 0.0)


def _layer_body_mix(mix_out_first, adj_ref, h_ref, h0_ref, m_ref, other_ref,
                    w_ref, o_ref, mix_ref):
    hi = jnp.dot(adj_ref[0], h_ref[...], preferred_element_type=jnp.float32)
    support = (1.0 - ALPHA) * hi + ALPHA * h0_ref[...]
    z = jnp.dot(support, m_ref[...], preferred_element_type=jnp.float32)
    out = jnp.maximum(jnp.tanh(z), 0.0)
    o_ref[...] = out
    w = w_ref[0, 0]
    if mix_out_first:
        mix_ref[...] = w * out + (1.0 - w) * other_ref[...]
    else:
        mix_ref[...] = w * other_ref[...] + (1.0 - w) * out


def _layer_stage(adj, k, H, h0, M, other=None, w2d=None, mix_out_first=False):
    grid = (N // BM,)
    adj_spec = pl.BlockSpec((1, BM, N), lambda i: (k, i, 0))
    h_spec = pl.BlockSpec((N, NH), lambda i: (0, 0))
    tile_spec = pl.BlockSpec((BM, NH), lambda i: (i, 0))
    m_spec = pl.BlockSpec((NH, NH), lambda i: (0, 0))
    tile_shape = jax.ShapeDtypeStruct((N, NH), jnp.float32)
    params = pltpu.CompilerParams(dimension_semantics=("parallel",),
                                  vmem_limit_bytes=100 * 1024 * 1024)
    if other is None:
        return pl.pallas_call(
            _layer_body_plain,
            grid=grid,
            in_specs=[adj_spec, h_spec, tile_spec, m_spec],
            out_specs=tile_spec,
            out_shape=tile_shape,
            compiler_params=params,
        )(adj, H, h0, M)
    import functools
    body = functools.partial(_layer_body_mix, mix_out_first)
    return pl.pallas_call(
        body,
        grid=grid,
        in_specs=[adj_spec, h_spec, tile_spec, m_spec, tile_spec,
                  pl.BlockSpec(memory_space=pltpu.SMEM)],
        out_specs=(tile_spec, tile_spec),
        out_shape=(tile_shape, tile_shape),
        compiler_params=params,
    )(adj, H, h0, M, other, w2d)


def _final_body(o00_ref, o01_ref, o10_ref, o11_ref, wt_ref, b_ref,
                fin_ref, mean_ref, logs_ref):
    s0 = o00_ref[...] + o01_ref[...]
    s1 = o10_ref[...] + o11_ref[...]
    wt = wt_ref[...]
    b = b_ref[...]
    l0 = jnp.dot(s0, wt, preferred_element_type=jnp.float32) + b
    l1 = jnp.dot(s1, wt, preferred_element_type=jnp.float32) + b

    def logsoftmax(z):
        m = jnp.max(z, axis=1, keepdims=True)
        e = z - m
        return e - jnp.log(jnp.sum(jnp.exp(e), axis=1, keepdims=True))

    ls0 = logsoftmax(l0)
    ls1 = logsoftmax(l1)
    fin_ref[...] = logsoftmax(l0 + l1)
    mean_ref[...] = 0.5 * (ls0 + ls1)
    logs_ref[0] = ls0
    logs_ref[1] = ls1


def _final_stage(o00, o01, o10, o11, fco_Wt, fco_b2):
    tile = pl.BlockSpec((N, NH), lambda: (0, 0))
    return pl.pallas_call(
        _final_body,
        in_specs=[tile, tile, tile, tile,
                  pl.BlockSpec((NH, NCLASS), lambda: (0, 0)),
                  pl.BlockSpec((1, NCLASS), lambda: (0, 0))],
        out_specs=(pl.BlockSpec((N, NCLASS), lambda: (0, 0)),
                   pl.BlockSpec((N, NCLASS), lambda: (0, 0)),
                   pl.BlockSpec((K, N, NCLASS), lambda: (0, 0, 0))),
        out_shape=(jax.ShapeDtypeStruct((N, NCLASS), jnp.float32),
                   jax.ShapeDtypeStruct((N, NCLASS), jnp.float32),
                   jax.ShapeDtypeStruct((K, N, NCLASS), jnp.float32)),
        compiler_params=pltpu.CompilerParams(
            vmem_limit_bytes=100 * 1024 * 1024),
    )(o00, o01, o10, o11, fco_Wt, fco_b2)


def kernel(x, adj, conv_W, fc_W, fc_b, fco_W, fco_b, w):
    fc_Wt = jnp.swapaxes(fc_W, 1, 2)
    fc_b3 = fc_b[:, None, :]
    fco_Wt = fco_W.T
    fco_b2 = fco_b[None, :]
    w2d = w.reshape(1, 1)

    fc = _fc_stage(x, fc_Wt, fc_b3)
    M = _ortho_stage(conv_W)
    h00 = fc[0]
    h01 = fc[1]
    M0 = M[0]
    M1 = M[1]

    out00, mix10 = _layer_stage(adj, 0, h00, h00, M0,
                                other=h01, w2d=w2d, mix_out_first=False)
    out01 = _layer_stage(adj, 0, out00, h00, M1)
    out10, mix11 = _layer_stage(adj, 1, mix10, h01, M0,
                                other=out01, w2d=w2d, mix_out_first=True)
    out11 = _layer_stage(adj, 1, mix11, h01, M1)

    fin, mean, logs = _final_stage(out00, out01, out10, out11,
                                   fco_Wt, fco_b2)
    return fin, mean, logs, w


# speedup vs baseline: 1.0798x; 1.0798x over previous
import math

import jax
import jax.numpy as jnp
from jax.experimental import pallas as pl
from jax.experimental.pallas import tpu as pltpu

K = 2
N = 10000
NFEAT = 128
NH = 64
NCLASS = 40
NLAYERS = 2
LAMDA = 0.5
ALPHA = 0.1

BM = 400


def _fc_body(x_ref, wt_ref, b_ref, o_ref):
    acc = jnp.dot(x_ref[0], wt_ref[0], preferred_element_type=jnp.float32)
    o_ref[0] = jnp.maximum(acc + b_ref[0], 0.0)


def _fc_stage(x, fc_Wt, fc_b3):
    return pl.pallas_call(
        _fc_body,
        grid=(K,),
        in_specs=[
            pl.BlockSpec((1, N, NFEAT), lambda k: (k, 0, 0)),
            pl.BlockSpec((1, NFEAT, NH), lambda k: (k, 0, 0)),
            pl.BlockSpec((1, 1, NH), lambda k: (k, 0, 0)),
        ],
        out_specs=pl.BlockSpec((1, N, NH), lambda k: (k, 0, 0)),
        out_shape=jax.ShapeDtypeStruct((K, N, NH), jnp.float32),
        compiler_params=pltpu.CompilerParams(
            dimension_semantics=("arbitrary",)),
    )(x, fc_Wt, fc_b3)


def _chol_solve(W, rows, lanes, eye):
    A = jax.lax.dot_general(W, W, (((0,), (0,)), ((), ())),
                            preferred_element_type=jnp.float32)
    A = A + 1e-4 * eye
    one = jnp.float32(1.0)
    zero = jnp.float32(0.0)
    cols = []
    recips = []
    for k in range(NH):
        colv = jax.lax.slice(A, (0, k), (NH, k + 1))
        akk = jax.lax.slice(A, (k, k), (k + 1, k + 1))
        rowv = jax.lax.slice(A, (k, 0), (k + 1, NH))
        rinv = jax.lax.rsqrt(akk)
        recips.append(rinv)
        row_ge = jnp.where(rows >= k, one, zero)
        cols.append(colv * rinv * row_ge)
        if k < NH - 1:
            row_gt = jnp.where(rows > k, one, zero)
            lane_ge = jnp.where(lanes >= k, one, zero)
            A = A - (colv * row_gt) * (rowv * lane_ge * (1.0 / akk))
    L = jnp.concatenate(cols, axis=1)
    Lt = L.T
    X = jnp.zeros((NH, NH), jnp.float32)
    for j in range(NH):
        ltcol = jax.lax.slice(Lt, (0, j), (NH, j + 1))
        acc = jnp.dot(X, ltcol, preferred_element_type=jnp.float32)
        wcol = jax.lax.slice(W, (0, j), (NH, j + 1))
        xcol = (wcol - acc) * recips[j]
        X = X + xcol * jnp.where(lanes == j, one, zero)
    return X


def _ortho_body(w_ref, m_ref):
    t0 = math.log(LAMDA / 1.0 + 1.0)
    t1 = math.log(LAMDA / 2.0 + 1.0)
    rows = jax.lax.broadcasted_iota(jnp.int32, (NH, 1), 0)
    lanes = jax.lax.broadcasted_iota(jnp.int32, (1, NH), 1)
    eye = (rows == lanes).astype(jnp.float32)
    X0 = _chol_solve(w_ref[0], rows, lanes, eye)
    X1 = _chol_solve(w_ref[1], rows, lanes, eye)
    m_ref[0] = t0 * X0 + (1.0 - t0) * eye
    m_ref[1] = t1 * X1 + (1.0 - t1) * eye


def _ortho_stage(conv_W):
    return pl.pallas_call(
        _ortho_body,
        in_specs=[pl.BlockSpec((NLAYERS, NH, NH), lambda: (0, 0, 0))],
        out_specs=pl.BlockSpec((NLAYERS, NH, NH), lambda: (0, 0, 0)),
        out_shape=jax.ShapeDtypeStruct((NLAYERS, NH, NH), jnp.float32),
    )(conv_W)


def _layer_body_plain(adj_ref, h_ref, h0_ref, m_ref, o_ref):
    hi = jnp.dot(adj_ref[0], h_ref[...], preferred_element_type=jnp.float32)
    support = (1.0 - ALPHA) * hi + ALPHA * h0_ref[...]
    z = jnp.dot(support, m_ref[...], preferred_element_type=jnp.float32)
    o_ref[...] = jnp.maximum(jnp.tanh(z), 0.0)


def _layer_body_mix(mix_out_first, adj_ref, h_ref, h0_ref, m_ref, other_ref,
                    w_ref, o_ref, mix_ref):
    hi = jnp.dot(adj_ref[0], h_ref[...], preferred_element_type=jnp.float32)
    support = (1.0 - ALPHA) * hi + ALPHA * h0_ref[...]
    z = jnp.dot(support, m_ref[...], preferred_element_type=jnp.float32)
    out = jnp.maximum(jnp.tanh(z), 0.0)
    o_ref[...] = out
    w = w_ref[0, 0]
    if mix_out_first:
        mix_ref[...] = w * out + (1.0 - w) * other_ref[...]
    else:
        mix_ref[...] = w * other_ref[...] + (1.0 - w) * out


def _layer_stage(adj, k, H, h0, M, other=None, w2d=None, mix_out_first=False):
    grid = (N // BM,)
    adj_spec = pl.BlockSpec((1, BM, N), lambda i: (k, i, 0))
    h_spec = pl.BlockSpec((N, NH), lambda i: (0, 0))
    tile_spec = pl.BlockSpec((BM, NH), lambda i: (i, 0))
    m_spec = pl.BlockSpec((NH, NH), lambda i: (0, 0))
    tile_shape = jax.ShapeDtypeStruct((N, NH), jnp.float32)
    params = pltpu.CompilerParams(dimension_semantics=("parallel",),
                                  vmem_limit_bytes=100 * 1024 * 1024)
    if other is None:
        return pl.pallas_call(
            _layer_body_plain,
            grid=grid,
            in_specs=[adj_spec, h_spec, tile_spec, m_spec],
            out_specs=tile_spec,
            out_shape=tile_shape,
            compiler_params=params,
        )(adj, H, h0, M)
    import functools
    body = functools.partial(_layer_body_mix, mix_out_first)
    return pl.pallas_call(
        body,
        grid=grid,
        in_specs=[adj_spec, h_spec, tile_spec, m_spec, tile_spec,
                  pl.BlockSpec(memory_space=pltpu.SMEM)],
        out_specs=(tile_spec, tile_spec),
        out_shape=(tile_shape, tile_shape),
        compiler_params=params,
    )(adj, H, h0, M, other, w2d)


def _final_body(o00_ref, o01_ref, o10_ref, o11_ref, wt_ref, b_ref,
                fin_ref, mean_ref, logs_ref):
    s0 = o00_ref[...] + o01_ref[...]
    s1 = o10_ref[...] + o11_ref[...]
    wt = wt_ref[...]
    b = b_ref[...]
    l0 = jnp.dot(s0, wt, preferred_element_type=jnp.float32) + b
    l1 = jnp.dot(s1, wt, preferred_element_type=jnp.float32) + b

    def logsoftmax(z):
        m = jnp.max(z, axis=1, keepdims=True)
        e = z - m
        return e - jnp.log(jnp.sum(jnp.exp(e), axis=1, keepdims=True))

    ls0 = logsoftmax(l0)
    ls1 = logsoftmax(l1)
    fin_ref[...] = logsoftmax(l0 + l1)
    mean_ref[...] = 0.5 * (ls0 + ls1)
    logs_ref[0] = ls0
    logs_ref[1] = ls1


def _final_stage(o00, o01, o10, o11, fco_Wt, fco_b2):
    tile = pl.BlockSpec((N, NH), lambda: (0, 0))
    return pl.pallas_call(
        _final_body,
        in_specs=[tile, tile, tile, tile,
                  pl.BlockSpec((NH, NCLASS), lambda: (0, 0)),
                  pl.BlockSpec((1, NCLASS), lambda: (0, 0))],
        out_specs=(pl.BlockSpec((N, NCLASS), lambda: (0, 0)),
                   pl.BlockSpec((N, NCLASS), lambda: (0, 0)),
                   pl.BlockSpec((K, N, NCLASS), lambda: (0, 0, 0))),
        out_shape=(jax.ShapeDtypeStruct((N, NCLASS), jnp.float32),
                   jax.ShapeDtypeStruct((N, NCLASS), jnp.float32),
                   jax.ShapeDtypeStruct((K, N, NCLASS), jnp.float32)),
        compiler_params=pltpu.CompilerParams(
            vmem_limit_bytes=100 * 1024 * 1024),
    )(o00, o01, o10, o11, fco_Wt, fco_b2)


def kernel(x, adj, conv_W, fc_W, fc_b, fco_W, fco_b, w):
    fc_Wt = jnp.swapaxes(fc_W, 1, 2)
    fc_b3 = fc_b[:, None, :]
    fco_Wt = fco_W.T
    fco_b2 = fco_b[None, :]
    w2d = w.reshape(1, 1)

    fc = _fc_stage(x, fc_Wt, fc_b3)
    M = _ortho_stage(conv_W)
    h00 = fc[0]
    h01 = fc[1]
    M0 = M[0]
    M1 = M[1]

    out00, mix10 = _layer_stage(adj, 0, h00, h00, M0,
                                other=h01, w2d=w2d, mix_out_first=False)
    out01 = _layer_stage(adj, 0, out00, h00, M1)
    out10, mix11 = _layer_stage(adj, 1, mix10, h01, M0,
                                other=out01, w2d=w2d, mix_out_first=True)
    out11 = _layer_stage(adj, 1, mix11, h01, M1)

    fin, mean, logs = _final_stage(out00, out01, out10, out11,
                                   fco_Wt, fco_b2)
    return fin, mean, logs, w


# speedup vs baseline: 1.2296x; 1.1388x over previous
import math

import jax
import jax.numpy as jnp
from jax.experimental import pallas as pl
from jax.experimental.pallas import tpu as pltpu

K = 2
N = 10000
NFEAT = 128
NH = 64
NCLASS = 40
NLAYERS = 2
LAMDA = 0.5
ALPHA = 0.1

BM = 400
NT = N // BM
T0 = math.log(LAMDA / 1.0 + 1.0)
T1 = math.log(LAMDA / 2.0 + 1.0)


def _chol_solve(W, rows, lanes, eye):
    A = jax.lax.dot_general(W, W, (((0,), (0,)), ((), ())),
                            preferred_element_type=jnp.float32)
    A = A + 1e-4 * eye
    one = jnp.float32(1.0)
    zero = jnp.float32(0.0)
    cols = []
    recips = []
    for k in range(NH):
        colv = jax.lax.slice(A, (0, k), (NH, k + 1))
        akk = jax.lax.slice(A, (k, k), (k + 1, k + 1))
        rowv = jax.lax.slice(A, (k, 0), (k + 1, NH))
        rinv = jax.lax.rsqrt(akk)
        recips.append(rinv)
        row_ge = jnp.where(rows >= k, one, zero)
        cols.append(colv * rinv * row_ge)
        if k < NH - 1:
            row_gt = jnp.where(rows > k, one, zero)
            lane_ge = jnp.where(lanes >= k, one, zero)
            A = A - (colv * row_gt) * (rowv * lane_ge * (rinv * rinv))
    L = jnp.concatenate(cols, axis=1)
    Lt = L.T
    X = jnp.zeros((NH, NH), jnp.float32)
    for j in range(NH):
        ltcol = jax.lax.slice(Lt, (0, j), (NH, j + 1))
        acc = jnp.dot(X, ltcol, preferred_element_type=jnp.float32)
        wcol = jax.lax.slice(W, (0, j), (NH, j + 1))
        xcol = (wcol - acc) * recips[j]
        X = X + xcol * jnp.where(lanes == j, one, zero)
    return X


def _logsoftmax(z):
    m = jnp.max(z, axis=1, keepdims=True)
    e = z - m
    return e - jnp.log(jnp.sum(jnp.exp(e), axis=1, keepdims=True))


def _fc_body(x_ref, wt_ref, b_ref, o_ref):
    f0 = jnp.dot(x_ref[0], wt_ref[0],
                 preferred_element_type=jnp.float32) + b_ref[0]
    f1 = jnp.dot(x_ref[1], wt_ref[1],
                 preferred_element_type=jnp.float32) + b_ref[1]
    o_ref[...] = jnp.maximum(jnp.concatenate([f0, f1], axis=1), 0.0)


def _fc_stage(x, fc_Wt, fc_b3):
    return pl.pallas_call(
        _fc_body,
        in_specs=[
            pl.BlockSpec((K, N, NFEAT), lambda: (0, 0, 0)),
            pl.BlockSpec((K, NFEAT, NH), lambda: (0, 0, 0)),
            pl.BlockSpec((K, 1, NH), lambda: (0, 0, 0)),
        ],
        out_specs=pl.BlockSpec((N, K * NH), lambda: (0, 0)),
        out_shape=jax.ShapeDtypeStruct((N, K * NH), jnp.float32),
    )(x, fc_Wt, fc_b3)


_LO = slice(0, NH)
_HI = slice(NH, 2 * NH)


def _mega_body(adj_ref, fcpk_ref, convw_ref, fcowt_ref, fcob_ref, w_ref,
               fin_ref, mean_ref, logs_ref, p1, p2, p3, msc):
    t = pl.program_id(0)
    i = t % NT
    ds = pl.ds(i * BM, BM)
    w = w_ref[0, 0]
    a = jnp.float32(ALPHA)
    na = jnp.float32(1.0 - ALPHA)

    @pl.when(t == 0)
    def _prologue():
        rows = jax.lax.broadcasted_iota(jnp.int32, (NH, 1), 0)
        lanes = jax.lax.broadcasted_iota(jnp.int32, (1, NH), 1)
        eye = (rows == lanes).astype(jnp.float32)
        X0 = _chol_solve(convw_ref[0], rows, lanes, eye)
        msc[0] = T0 * X0 + (1.0 - T0) * eye
        X1 = _chol_solve(convw_ref[1], rows, lanes, eye)
        msc[1] = T1 * X1 + (1.0 - T1) * eye

    @pl.when(t < NT)
    def _l0():
        hi = jnp.dot(adj_ref[0], fcpk_ref[:, _LO],
                     preferred_element_type=jnp.float32)
        support = na * hi + a * fcpk_ref[ds, _LO]
        z = jnp.dot(support, msc[0], preferred_element_type=jnp.float32)
        out = jnp.maximum(jnp.tanh(z), 0.0)
        p1[ds, _LO] = out
        p2[ds, _LO] = w * fcpk_ref[ds, _HI] + (1.0 - w) * out

    @pl.when((t >= NT) & (t < 2 * NT))
    def _l1():
        hi = jnp.dot(adj_ref[0], p1[:, _LO],
                     preferred_element_type=jnp.float32)
        support = na * hi + a * fcpk_ref[ds, _LO]
        z = jnp.dot(support, msc[1], preferred_element_type=jnp.float32)
        p1[ds, _HI] = jnp.maximum(jnp.tanh(z), 0.0)

    @pl.when((t >= 2 * NT) & (t < 3 * NT))
    def _l2():
        hi = jnp.dot(adj_ref[0], p2[:, _LO],
                     preferred_element_type=jnp.float32)
        support = na * hi + a * fcpk_ref[ds, _HI]
        z = jnp.dot(support, msc[0], preferred_element_type=jnp.float32)
        out = jnp.maximum(jnp.tanh(z), 0.0)
        p2[ds, _HI] = out
        p3[ds, _LO] = w * out + (1.0 - w) * p1[ds, _HI]

    @pl.when(t >= 3 * NT)
    def _l3():
        hi = jnp.dot(adj_ref[0], p3[:, _LO],
                     preferred_element_type=jnp.float32)
        support = na * hi + a * fcpk_ref[ds, _HI]
        z = jnp.dot(support, msc[1], preferred_element_type=jnp.float32)
        out11 = jnp.maximum(jnp.tanh(z), 0.0)
        s0 = p1[ds, _LO] + p1[ds, _HI]
        s1 = p2[ds, _HI] + out11
        wt = fcowt_ref[...]
        b = fcob_ref[...]
        lg0 = jnp.dot(s0, wt, preferred_element_type=jnp.float32) + b
        lg1 = jnp.dot(s1, wt, preferred_element_type=jnp.float32) + b
        ls0 = _logsoftmax(lg0)
        ls1 = _logsoftmax(lg1)
        fin_ref[...] = _logsoftmax(lg0 + lg1)
        mean_ref[...] = 0.5 * (ls0 + ls1)
        logs_ref[0] = ls0
        logs_ref[1] = ls1


def kernel(x, adj, conv_W, fc_W, fc_b, fco_W, fco_b, w):
    fc_Wt = jnp.swapaxes(fc_W, 1, 2)
    fc_b3 = fc_b[:, None, :]
    fco_Wt = fco_W.T
    fco_b2 = fco_b[None, :]
    w2d = w.reshape(1, 1)

    fcpk = _fc_stage(x, fc_Wt, fc_b3)

    fin, mean, logs = pl.pallas_call(
        _mega_body,
        grid=(4 * NT,),
        in_specs=[
            pl.BlockSpec((1, BM, N), lambda t: (t // (2 * NT), t % NT, 0)),
            pl.BlockSpec((N, K * NH), lambda t: (0, 0)),
            pl.BlockSpec((NLAYERS, NH, NH), lambda t: (0, 0, 0)),
            pl.BlockSpec((NH, NCLASS), lambda t: (0, 0)),
            pl.BlockSpec((1, NCLASS), lambda t: (0, 0)),
            pl.BlockSpec(memory_space=pltpu.SMEM),
        ],
        out_specs=(
            pl.BlockSpec((BM, NCLASS),
                         lambda t: (jnp.where(t >= 3 * NT, t % NT, 0), 0)),
            pl.BlockSpec((BM, NCLASS),
                         lambda t: (jnp.where(t >= 3 * NT, t % NT, 0), 0)),
            pl.BlockSpec((K, BM, NCLASS),
                         lambda t: (0, jnp.where(t >= 3 * NT, t % NT, 0), 0)),
        ),
        out_shape=(
            jax.ShapeDtypeStruct((N, NCLASS), jnp.float32),
            jax.ShapeDtypeStruct((N, NCLASS), jnp.float32),
            jax.ShapeDtypeStruct((K, N, NCLASS), jnp.float32),
        ),
        scratch_shapes=[
            pltpu.VMEM((N, K * NH), jnp.float32),
            pltpu.VMEM((N, K * NH), jnp.float32),
            pltpu.VMEM((N, NH), jnp.float32),
            pltpu.VMEM((NLAYERS, NH, NH), jnp.float32),
        ],
        compiler_params=pltpu.CompilerParams(
            dimension_semantics=("arbitrary",),
            vmem_limit_bytes=62 * 1024 * 1024),
    )(adj, fcpk, conv_W, fco_Wt, fco_b2, w2d)
    return fin, mean, logs, w
